# Initial kernel scaffold; baseline (speedup 1.0000x reference)
import jax
import jax.numpy as jnp
from jax import lax
from jax.experimental import pallas as pl
from jax.experimental.pallas import tpu as pltpu
from jax.experimental.pallas import tpu_sc as plsc

N = 10000
E = 320000
D = 128
DE = 16
K = 8
G = 128
T = 10

NC = 2
NS = 16
CHUNK = 80
EPT = E // (NC * NS)
NCHUNK = EPT // CHUNK
RPT = N // NS
XZ = 125


def _isnan0(v):
    return jnp.where(jnp.isnan(v), jnp.float32(0.0), v)


def _sqrt_sc(z):
    i = plsc.bitcast(z, jnp.int32)
    i = jnp.int32(0x5F3759DF) - (i >> 1)
    y = plsc.bitcast(i, jnp.float32)
    for _ in range(3):
        y = y * (jnp.float32(1.5) - jnp.float32(0.5) * z * y * y)
    return z * y


def _sc_body(x_hbm, u_hbm, src_hbm, dst_hbm, ea_hbm,
             out_x, out_ea, out_pe,
             acc_x, acc_ea, acc_pe,
             src_idx, dst_idx, x_rows, ea_buf, pe_buf, us, ud,
             zb_x, zb_s, sem):
    cid = lax.axis_index("c")
    sid = lax.axis_index("s")
    wid = cid * NS + sid

    zero16 = jnp.zeros((16,), jnp.float32)

    def zrow_x(r, carry):
        for cc in range(D // 16):
            zb_x[r, pl.ds(cc * 16, 16)] = zero16
        return carry
    lax.fori_loop(0, XZ, zrow_x, None)

    def zrow_s(r, carry):
        zb_s[r, :] = zero16
        return carry
    lax.fori_loop(0, RPT, zrow_s, None)

    def zrow_pe(r, carry):
        pe_buf[r, :] = zero16
        return carry
    lax.fori_loop(0, CHUNK, zrow_pe, None)

    for j in range(RPT // XZ):
        pltpu.sync_copy(zb_x, acc_x.at[pl.ds(sid * RPT + j * XZ, XZ)])
    pltpu.sync_copy(zb_s, acc_ea.at[pl.ds(sid * RPT, RPT)])
    pltpu.sync_copy(zb_s, acc_pe.at[pl.ds(sid * RPT, RPT)])
    plsc.subcore_barrier()

    lanes = lax.iota(jnp.int32, 16)
    col0 = jnp.zeros((16,), jnp.int32)

    def chunk_body(c, carry):
        e0 = wid * EPT + c * CHUNK
        pltpu.sync_copy(src_hbm.at[pl.ds(e0, CHUNK)], src_idx)
        pltpu.sync_copy(dst_hbm.at[pl.ds(e0, CHUNK)], dst_idx)
        pltpu.sync_copy(ea_hbm.at[pl.ds(e0, CHUNK)], ea_buf)
        cx = pltpu.async_copy(x_hbm.at[src_idx], x_rows, sem)
        cs = pltpu.async_copy(u_hbm.at[src_idx], us, sem)
        cd = pltpu.async_copy(u_hbm.at[dst_idx], ud, sem)
        cx.wait()
        cs.wait()
        cd.wait()

        for g in range(CHUNK // 16):
            rows = lanes + g * 16
            acc = jnp.zeros((16,), jnp.float32)
            for k in range(K):
                kk = jnp.full((16,), k, jnp.int32)
                a = _isnan0(plsc.load_gather(us, [rows, kk]))
                b = _isnan0(plsc.load_gather(ud, [rows, kk]))
                d = a - b
                acc = acc + d * d
            pe = _sqrt_sc(acc + jnp.float32(1e-12))
            plsc.store_scatter(pe_buf, [rows, col0], pe)

        pltpu.sync_copy(x_rows, acc_x.at[dst_idx], add=True)
        pltpu.sync_copy(ea_buf, acc_ea.at[dst_idx], add=True)
        pltpu.sync_copy(pe_buf, acc_pe.at[dst_idx], add=True)
        return carry

    lax.fori_loop(0, NCHUNK, chunk_body, None)
    plsc.subcore_barrier()

    base = cid * N + sid * RPT
    for j in range(RPT // XZ):
        pltpu.sync_copy(acc_x.at[pl.ds(sid * RPT + j * XZ, XZ)], zb_x)
        pltpu.sync_copy(zb_x, out_x.at[pl.ds(base + j * XZ, XZ)])
    pltpu.sync_copy(acc_ea.at[pl.ds(sid * RPT, RPT)], zb_s)
    pltpu.sync_copy(zb_s, out_ea.at[pl.ds(base, RPT)])
    pltpu.sync_copy(acc_pe.at[pl.ds(sid * RPT, RPT)], zb_s)
    pltpu.sync_copy(zb_s, out_pe.at[pl.ds(base, RPT)])


_sc_edges = pl.kernel(
    _sc_body,
    out_type=(
        jax.ShapeDtypeStruct((NC * N, D), jnp.float32),
        jax.ShapeDtypeStruct((NC * N, DE), jnp.float32),
        jax.ShapeDtypeStruct((NC * N, DE), jnp.float32),
    ),
    mesh=plsc.VectorSubcoreMesh(core_axis_name="c", subcore_axis_name="s"),
    scratch_types=(
        pltpu.VMEM_SHARED((N, D), jnp.float32),
        pltpu.VMEM_SHARED((N, DE), jnp.float32),
        pltpu.VMEM_SHARED((N, DE), jnp.float32),
        pltpu.VMEM((CHUNK,), jnp.int32),
        pltpu.VMEM((CHUNK,), jnp.int32),
        pltpu.VMEM((CHUNK, D), jnp.float32),
        pltpu.VMEM((CHUNK, DE), jnp.float32),
        pltpu.VMEM((CHUNK, DE), jnp.float32),
        pltpu.VMEM((CHUNK, K), jnp.float32),
        pltpu.VMEM((CHUNK, K), jnp.float32),
        pltpu.VMEM((XZ, D), jnp.float32),
        pltpu.VMEM((RPT, DE), jnp.float32),
        pltpu.SemaphoreType.DMA,
    ),
)


ROWS = 1000
NB = N // ROWS


def _tc_body(x_ref, px0, px1, pea0, pea1, ppe0, ppe1, s_ref, batch_ref,
             we_ref, bpe_ref, ws_ref, wn_ref, wo_ref, bo_ref,
             out_ref, gsum, gcnt):
    i = pl.program_id(0)

    @pl.when(i == 0)
    def _init():
        gsum[...] = jnp.zeros((G, D), jnp.float32)
        gcnt[...] = jnp.zeros((G, 1), jnp.float32)

    agg = (px0[...] + px1[...]
           + jnp.dot(pea0[...] + pea1[...], we_ref[...],
                     preferred_element_type=jnp.float32)
           + (ppe0[..., 0:1] + ppe1[..., 0:1]) * bpe_ref[...])
    pre = (jnp.dot(x_ref[...], ws_ref[...], preferred_element_type=jnp.float32)
           + jnp.dot(agg, wn_ref[...], preferred_element_type=jnp.float32))
    enc = jnp.maximum(pre * s_ref[...], 0.0)

    b2 = jnp.reshape(batch_ref[...], (1, ROWS))
    oht = (lax.broadcasted_iota(jnp.int32, (G, ROWS), 0) == b2
           ).astype(jnp.float32)
    gsum[...] += jnp.dot(oht, enc, preferred_element_type=jnp.float32)
    gcnt[...] += jnp.sum(oht, axis=1, keepdims=True)

    @pl.when(i == NB - 1)
    def _fin():
        rep = gsum[...] / jnp.maximum(gcnt[...], 1.0)
        out_ref[...] = (jnp.dot(rep, wo_ref[...],
                                preferred_element_type=jnp.float32)
                        + bo_ref[...])


def _row_spec(cols):
    return pl.BlockSpec((ROWS, cols), lambda i: (i, 0))


def _full_spec(r, c):
    return pl.BlockSpec((r, c), lambda i: (0, 0))


_tc_dense = pl.pallas_call(
    _tc_body,
    grid=(NB,),
    in_specs=[
        _row_spec(D), _row_spec(D), _row_spec(D),
        _row_spec(DE), _row_spec(DE), _row_spec(DE), _row_spec(DE),
        _row_spec(1),
        pl.BlockSpec((1, 1, ROWS), lambda i: (i, 0, 0)),
        _full_spec(DE, D), _full_spec(1, D), _full_spec(D, D),
        _full_spec(D, D), _full_spec(D, T), _full_spec(1, T),
    ],
    out_specs=pl.BlockSpec((G, T), lambda i: (0, 0)),
    out_shape=jax.ShapeDtypeStruct((G, T), jnp.float32),
    scratch_shapes=[
        pltpu.VMEM((G, D), jnp.float32),
        pltpu.VMEM((G, 1), jnp.float32),
    ],
)


@jax.jit
def kernel(x, edge_index, edge_attr, snorm_n, EigVals, EigVecs, batch,
           W_edge, b_pe, W_self, W_nbr, W_out, b_out):
    src = edge_index[0]
    dst = edge_index[1]
    part_x, part_ea, part_pe = _sc_edges(x, EigVecs, src, dst, edge_attr)
    batch3 = jnp.reshape(batch, (NB, 1, ROWS))
    return _tc_dense(
        x, part_x[:N], part_x[N:],
        part_ea[:N], part_ea[N:],
        part_pe[:N], part_pe[N:],
        snorm_n, batch3,
        W_edge, jnp.reshape(b_pe, (1, D)), W_self, W_nbr,
        W_out, jnp.reshape(b_out, (1, T)),
    )


# speedup vs baseline: 5.4878x; 5.4878x over previous
import jax
import jax.numpy as jnp
from jax import lax
from jax.experimental import pallas as pl
from jax.experimental.pallas import tpu as pltpu
from jax.experimental.pallas import tpu_sc as plsc

N = 10000
E = 320000
D = 128
DE = 16
K = 8
G = 128
T = 10

NC = 2
NS = 16
CHUNK = 80
EPT = E // (NC * NS)
NCHUNK = EPT // CHUNK
NPAD = 10240
RPT = NPAD // NS
XZ = 64
SZ = 160


def _isnan0(v):
    return jnp.where(jnp.isnan(v), jnp.float32(0.0), v)


def _sqrt_sc(z):
    i = plsc.bitcast(z, jnp.int32)
    i = jnp.int32(0x5F3759DF) - (i >> 1)
    y = plsc.bitcast(i, jnp.float32)
    for _ in range(3):
        y = y * (jnp.float32(1.5) - jnp.float32(0.5) * z * y * y)
    return z * y


def _sc_body(x_hbm, u_hbm, src_hbm, dst_hbm, ea_hbm,
             out_x, out_ea, out_pe,
             acc_x, acc_ea,
             src_idx, dst_idx, x_rows, ea_buf, us, ud, pe_acc,
             zb_x, zb_s, sem, sem_s, sem_d):
    cid = lax.axis_index("c")
    sid = lax.axis_index("s")
    wid = cid * NS + sid

    zero16 = jnp.zeros((16,), jnp.float32)

    def zrow_x(r, carry):
        for cc in range(D // 16):
            zb_x[r, pl.ds(cc * 16, 16)] = zero16
        return carry
    lax.fori_loop(0, XZ, zrow_x, None)

    def zrow_s(r, carry):
        zb_s[r, :] = zero16
        return carry
    lax.fori_loop(0, SZ, zrow_s, None)

    def zrow_pe(r, carry):
        pe_acc[pl.ds(r * 16, 16)] = zero16
        return carry
    lax.fori_loop(0, N // 16, zrow_pe, None)

    for j in range(RPT // XZ):
        pltpu.sync_copy(zb_x, acc_x.at[pl.ds(sid * RPT + j * XZ, XZ)])
    for j in range(RPT // SZ):
        pltpu.sync_copy(zb_s, acc_ea.at[pl.ds(sid * RPT + j * SZ, SZ)])
    plsc.subcore_barrier()

    lanes = lax.iota(jnp.int32, 16)

    def chunk_body(c, carry):
        e0 = wid * EPT + c * CHUNK
        pltpu.sync_copy(src_hbm.at[pl.ds(e0, CHUNK)], src_idx)
        pltpu.sync_copy(dst_hbm.at[pl.ds(e0, CHUNK)], dst_idx)
        pltpu.sync_copy(ea_hbm.at[pl.ds(e0, CHUNK)], ea_buf)
        cx = pltpu.async_copy(x_hbm.at[src_idx], x_rows, sem)
        cs = pltpu.async_copy(u_hbm.at[src_idx], us, sem_s)
        cd = pltpu.async_copy(u_hbm.at[dst_idx], ud, sem_d)
        pltpu.sync_copy(ea_buf, acc_ea.at[dst_idx], add=True)
        cs.wait()
        cd.wait()

        for g in range(CHUNK // 16):
            rows = lanes + g * 16
            acc = jnp.zeros((16,), jnp.float32)
            for k in range(K):
                kk = jnp.full((16,), k, jnp.int32)
                a = _isnan0(plsc.load_gather(us, [rows, kk]))
                b = _isnan0(plsc.load_gather(ud, [rows, kk]))
                d = a - b
                acc = acc + d * d
            pe = _sqrt_sc(acc + jnp.float32(1e-12))
            dv = dst_idx[pl.ds(g * 16, 16)]
            plsc.addupdate_scatter(pe_acc, [dv], pe)

        cx.wait()
        pltpu.sync_copy(x_rows, acc_x.at[dst_idx], add=True)
        return carry

    lax.fori_loop(0, NCHUNK, chunk_body, None)
    plsc.subcore_barrier()

    base = cid * NPAD + sid * RPT
    for j in range(RPT // XZ):
        pltpu.sync_copy(acc_x.at[pl.ds(sid * RPT + j * XZ, XZ)], zb_x)
        pltpu.sync_copy(zb_x, out_x.at[pl.ds(base + j * XZ, XZ)])
    for j in range(RPT // SZ):
        pltpu.sync_copy(acc_ea.at[pl.ds(sid * RPT + j * SZ, SZ)], zb_s)
        pltpu.sync_copy(zb_s, out_ea.at[pl.ds(base + j * SZ, SZ)])
    pltpu.sync_copy(pe_acc, out_pe.at[pl.ds(wid * N, N)])


_sc_edges = pl.kernel(
    _sc_body,
    out_type=(
        jax.ShapeDtypeStruct((NC * NPAD, D), jnp.float32),
        jax.ShapeDtypeStruct((NC * NPAD, DE), jnp.float32),
        jax.ShapeDtypeStruct((NC * NS * N,), jnp.float32),
    ),
    mesh=plsc.VectorSubcoreMesh(core_axis_name="c", subcore_axis_name="s"),
    compiler_params=pltpu.CompilerParams(
        needs_layout_passes=False, use_tc_tiling_on_sc=False),
    scratch_types=(
        pltpu.VMEM_SHARED((NPAD, D), jnp.float32),
        pltpu.VMEM_SHARED((NPAD, DE), jnp.float32),
        pltpu.VMEM((CHUNK,), jnp.int32),
        pltpu.VMEM((CHUNK,), jnp.int32),
        pltpu.VMEM((CHUNK, D), jnp.float32),
        pltpu.VMEM((CHUNK, DE), jnp.float32),
        pltpu.VMEM((CHUNK, 2 * K), jnp.float32),
        pltpu.VMEM((CHUNK, 2 * K), jnp.float32),
        pltpu.VMEM((N,), jnp.float32),
        pltpu.VMEM((XZ, D), jnp.float32),
        pltpu.VMEM((SZ, DE), jnp.float32),
        pltpu.SemaphoreType.DMA,
        pltpu.SemaphoreType.DMA,
        pltpu.SemaphoreType.DMA,
    ),
)


ROWS = 1000
NB = N // ROWS


def _tc_body(x_ref, px0, px1, pea0, pea1, ppe_ref, s_ref, batch_ref,
             we_ref, bpe_ref, ws_ref, wn_ref, wo_ref, bo_ref,
             out_ref, gsum, gcnt):
    i = pl.program_id(0)

    @pl.when(i == 0)
    def _init():
        gsum[...] = jnp.zeros((G, D), jnp.float32)
        gcnt[...] = jnp.zeros((G, 1), jnp.float32)

    agg = (px0[...] + px1[...]
           + jnp.dot(pea0[...] + pea1[...], we_ref[...],
                     preferred_element_type=jnp.float32,
                     precision=lax.Precision.HIGHEST)
           + jnp.sum(ppe_ref[...], axis=1, keepdims=True) * bpe_ref[...])
    pre = (jnp.dot(x_ref[...], ws_ref[...], preferred_element_type=jnp.float32,
                     precision=lax.Precision.HIGHEST)
           + jnp.dot(agg, wn_ref[...], preferred_element_type=jnp.float32,
                     precision=lax.Precision.HIGHEST))
    enc = jnp.maximum(pre * s_ref[...], 0.0)

    b2 = jnp.reshape(batch_ref[...], (1, ROWS))
    oht = (lax.broadcasted_iota(jnp.int32, (G, ROWS), 0) == b2
           ).astype(jnp.float32)
    gsum[...] += jnp.dot(oht, enc, preferred_element_type=jnp.float32,
                     precision=lax.Precision.HIGHEST)
    gcnt[...] += jnp.sum(oht, axis=1, keepdims=True)

    @pl.when(i == NB - 1)
    def _fin():
        rep = gsum[...] / jnp.maximum(gcnt[...], 1.0)
        out_ref[...] = (jnp.dot(rep, wo_ref[...],
                                preferred_element_type=jnp.float32,
                     precision=lax.Precision.HIGHEST)
                        + bo_ref[...])


def _row_spec(cols):
    return pl.BlockSpec((ROWS, cols), lambda i: (i, 0))


def _full_spec(r, c):
    return pl.BlockSpec((r, c), lambda i: (0, 0))


_tc_dense = pl.pallas_call(
    _tc_body,
    grid=(NB,),
    in_specs=[
        _row_spec(D), _row_spec(D), _row_spec(D),
        _row_spec(DE), _row_spec(DE), _row_spec(NC * NS),
        _row_spec(1),
        pl.BlockSpec((1, 1, ROWS), lambda i: (i, 0, 0)),
        _full_spec(DE, D), _full_spec(1, D), _full_spec(D, D),
        _full_spec(D, D), _full_spec(D, T), _full_spec(1, T),
    ],
    out_specs=pl.BlockSpec((G, T), lambda i: (0, 0)),
    out_shape=jax.ShapeDtypeStruct((G, T), jnp.float32),
    scratch_shapes=[
        pltpu.VMEM((G, D), jnp.float32),
        pltpu.VMEM((G, 1), jnp.float32),
    ],
)


@jax.jit
def kernel(x, edge_index, edge_attr, snorm_n, EigVals, EigVecs, batch,
           W_edge, b_pe, W_self, W_nbr, W_out, b_out):
    src = edge_index[0]
    dst = edge_index[1]
    u_pad = jnp.pad(EigVecs, ((0, 0), (0, K)))
    part_x, part_ea, part_pe = _sc_edges(x, u_pad, src, dst, edge_attr)
    batch3 = jnp.reshape(batch, (NB, 1, ROWS))
    return _tc_dense(
        x, part_x[:N], part_x[NPAD:NPAD + N],
        part_ea[:N], part_ea[NPAD:NPAD + N],
        jnp.transpose(jnp.reshape(part_pe, (NC * NS, N))),
        snorm_n, batch3,
        W_edge, jnp.reshape(b_pe, (1, D)), W_self, W_nbr,
        W_out, jnp.reshape(b_out, (1, T)),
    )


# speedup vs baseline: 5.6806x; 1.0351x over previous
import jax
import jax.numpy as jnp
from jax import lax
from jax.experimental import pallas as pl
from jax.experimental.pallas import tpu as pltpu
from jax.experimental.pallas import tpu_sc as plsc

N = 10000
E = 320000
D = 128
DE = 16
K = 8
G = 128
T = 10

NC = 2
NS = 16
CHUNK = 80
EPT = E // (NC * NS)
NCHUNK = EPT // CHUNK
NPAD = 10240
RPT = NPAD // NS
XZ = 64
SZ = 160


def _isnan0(v):
    return jnp.where(jnp.isnan(v), jnp.float32(0.0), v)


def _sqrt_sc(z):
    i = plsc.bitcast(z, jnp.int32)
    i = jnp.int32(0x5F3759DF) - (i >> 1)
    y = plsc.bitcast(i, jnp.float32)
    for _ in range(3):
        y = y * (jnp.float32(1.5) - jnp.float32(0.5) * z * y * y)
    return z * y


def _sc_body(x_hbm, u_hbm, src_hbm, dst_hbm, ea_hbm,
             out_x, out_ea, out_pe,
             acc_x, acc_ea,
             src_idx, dst_idx, x_rows, ea_buf, us, ud, pe_acc,
             zb_x, zb_s, sem, sem_s, sem_d):
    cid = lax.axis_index("c")
    sid = lax.axis_index("s")
    wid = cid * NS + sid

    zero16 = jnp.zeros((16,), jnp.float32)

    def zrow_x(r, carry):
        for cc in range(D // 16):
            zb_x[r, pl.ds(cc * 16, 16)] = zero16
        return carry
    lax.fori_loop(0, XZ, zrow_x, None)

    def zrow_s(r, carry):
        zb_s[r, :] = zero16
        return carry
    lax.fori_loop(0, SZ, zrow_s, None)

    def zrow_pe(r, carry):
        pe_acc[pl.ds(r * 16, 16)] = zero16
        return carry
    lax.fori_loop(0, N // 16, zrow_pe, None)

    for j in range(RPT // XZ):
        pltpu.sync_copy(zb_x, acc_x.at[pl.ds(sid * RPT + j * XZ, XZ)])
    for j in range(RPT // SZ):
        pltpu.sync_copy(zb_s, acc_ea.at[pl.ds(sid * RPT + j * SZ, SZ)])
    plsc.subcore_barrier()

    lanes = lax.iota(jnp.int32, 16)

    def chunk_body(c, carry):
        e0 = wid * EPT + c * CHUNK
        pltpu.sync_copy(src_hbm.at[pl.ds(e0, CHUNK)], src_idx)
        pltpu.sync_copy(dst_hbm.at[pl.ds(e0, CHUNK)], dst_idx)
        pltpu.sync_copy(ea_hbm.at[pl.ds(e0, CHUNK)], ea_buf)
        cx = pltpu.async_copy(x_hbm.at[src_idx], x_rows, sem)
        cs = pltpu.async_copy(u_hbm.at[src_idx], us, sem_s)
        cd = pltpu.async_copy(u_hbm.at[dst_idx], ud, sem_d)
        pltpu.sync_copy(ea_buf, acc_ea.at[dst_idx], add=True)
        cs.wait()
        cd.wait()

        for g in range(CHUNK // 16):
            rows = lanes + g * 16
            acc = jnp.zeros((16,), jnp.float32)
            for k in range(K):
                kk = jnp.full((16,), k, jnp.int32)
                a = _isnan0(plsc.load_gather(us, [rows, kk]))
                b = _isnan0(plsc.load_gather(ud, [rows, kk]))
                d = a - b
                acc = acc + d * d
            pe = _sqrt_sc(acc + jnp.float32(1e-12))
            dv = dst_idx[pl.ds(g * 16, 16)]
            plsc.addupdate_scatter(pe_acc, [dv], pe)

        cx.wait()
        pltpu.sync_copy(x_rows, acc_x.at[dst_idx], add=True)
        return carry

    lax.fori_loop(0, NCHUNK, chunk_body, None)
    plsc.subcore_barrier()

    base = cid * NPAD + sid * RPT
    for j in range(RPT // XZ):
        pltpu.sync_copy(acc_x.at[pl.ds(sid * RPT + j * XZ, XZ)], zb_x)
        pltpu.sync_copy(zb_x, out_x.at[pl.ds(base + j * XZ, XZ)])
    for j in range(RPT // SZ):
        pltpu.sync_copy(acc_ea.at[pl.ds(sid * RPT + j * SZ, SZ)], zb_s)
        pltpu.sync_copy(zb_s, out_ea.at[pl.ds(base + j * SZ, SZ)])
    pltpu.sync_copy(pe_acc, out_pe.at[pl.ds(wid * N, N)])


_sc_edges = pl.kernel(
    _sc_body,
    out_type=(
        jax.ShapeDtypeStruct((NC * NPAD, D), jnp.float32),
        jax.ShapeDtypeStruct((NC * NPAD, DE), jnp.float32),
        jax.ShapeDtypeStruct((NC * NS * N,), jnp.float32),
    ),
    mesh=plsc.VectorSubcoreMesh(core_axis_name="c", subcore_axis_name="s"),
    compiler_params=pltpu.CompilerParams(
        needs_layout_passes=False, use_tc_tiling_on_sc=False),
    scratch_types=(
        pltpu.VMEM_SHARED((NPAD, D), jnp.float32),
        pltpu.VMEM_SHARED((NPAD, DE), jnp.float32),
        pltpu.VMEM((CHUNK,), jnp.int32),
        pltpu.VMEM((CHUNK,), jnp.int32),
        pltpu.VMEM((CHUNK, D), jnp.float32),
        pltpu.VMEM((CHUNK, DE), jnp.float32),
        pltpu.VMEM((CHUNK, 2 * K), jnp.float32),
        pltpu.VMEM((CHUNK, 2 * K), jnp.float32),
        pltpu.VMEM((N,), jnp.float32),
        pltpu.VMEM((XZ, D), jnp.float32),
        pltpu.VMEM((SZ, DE), jnp.float32),
        pltpu.SemaphoreType.DMA,
        pltpu.SemaphoreType.DMA,
        pltpu.SemaphoreType.DMA,
    ),
)


ROWS = 1024
NB = NPAD // ROWS


def _tc_body(x_ref, px0, px1, pea0, pea1, ppe_ref, s_ref, batch_ref,
             we_ref, bpe_ref, ws_ref, wn_ref, wo_ref, bo_ref,
             out_ref, gsum, gcnt):
    i = pl.program_id(0)

    @pl.when(i == 0)
    def _init():
        gsum[...] = jnp.zeros((G, D), jnp.float32)
        gcnt[...] = jnp.zeros((G, 1), jnp.float32)

    agg = (px0[...] + px1[...]
           + jnp.dot(pea0[...] + pea1[...], we_ref[...],
                     preferred_element_type=jnp.float32,
                     precision=lax.Precision.HIGHEST)
           + jnp.sum(ppe_ref[...], axis=1, keepdims=True) * bpe_ref[...])
    pre = (jnp.dot(x_ref[...], ws_ref[...], preferred_element_type=jnp.float32,
                     precision=lax.Precision.HIGHEST)
           + jnp.dot(agg, wn_ref[...], preferred_element_type=jnp.float32,
                     precision=lax.Precision.HIGHEST))
    enc = jnp.maximum(pre * s_ref[...], 0.0)

    b2 = jnp.reshape(batch_ref[...], (1, ROWS))
    oht = (lax.broadcasted_iota(jnp.int32, (G, ROWS), 0) == b2
           ).astype(jnp.float32)
    gsum[...] += jnp.dot(oht, enc, preferred_element_type=jnp.float32,
                     precision=lax.Precision.HIGHEST)
    gcnt[...] += jnp.sum(oht, axis=1, keepdims=True)

    @pl.when(i == NB - 1)
    def _fin():
        rep = gsum[...] / jnp.maximum(gcnt[...], 1.0)
        out_ref[...] = (jnp.dot(rep, wo_ref[...],
                                preferred_element_type=jnp.float32,
                     precision=lax.Precision.HIGHEST)
                        + bo_ref[...])


def _row_spec(cols):
    return pl.BlockSpec((ROWS, cols), lambda i: (i, 0))


def _row_spec2(cols):
    return pl.BlockSpec((ROWS, cols), lambda i: (i + NB, 0))


def _full_spec(r, c):
    return pl.BlockSpec((r, c), lambda i: (0, 0))


_tc_dense = pl.pallas_call(
    _tc_body,
    grid=(NB,),
    in_specs=[
        _row_spec(D), _row_spec(D), _row_spec2(D),
        _row_spec(DE), _row_spec2(DE), _row_spec(NC * NS),
        _row_spec(1),
        pl.BlockSpec((1, 1, ROWS), lambda i: (i, 0, 0)),
        _full_spec(DE, D), _full_spec(1, D), _full_spec(D, D),
        _full_spec(D, D), _full_spec(D, T), _full_spec(1, T),
    ],
    out_specs=pl.BlockSpec((G, T), lambda i: (0, 0)),
    out_shape=jax.ShapeDtypeStruct((G, T), jnp.float32),
    scratch_shapes=[
        pltpu.VMEM((G, D), jnp.float32),
        pltpu.VMEM((G, 1), jnp.float32),
    ],
)


@jax.jit
def kernel(x, edge_index, edge_attr, snorm_n, EigVals, EigVecs, batch,
           W_edge, b_pe, W_self, W_nbr, W_out, b_out):
    src = edge_index[0]
    dst = edge_index[1]
    u_pad = jnp.pad(EigVecs, ((0, 0), (0, K)))
    part_x, part_ea, part_pe = _sc_edges(x, u_pad, src, dst, edge_attr)
    pad_n = NPAD - N
    x_p = jnp.pad(x, ((0, pad_n), (0, 0)))
    s_p = jnp.pad(snorm_n, ((0, pad_n), (0, 0)))
    pe_t = jnp.pad(jnp.transpose(jnp.reshape(part_pe, (NC * NS, N))),
                   ((0, pad_n), (0, 0)))
    batch3 = jnp.reshape(
        jnp.pad(batch, (0, pad_n), constant_values=G), (NB, 1, ROWS))
    return _tc_dense(
        x_p, part_x, part_x,
        part_ea, part_ea, pe_t,
        s_p, batch3,
        W_edge, jnp.reshape(b_pe, (1, D)), W_self, W_nbr,
        W_out, jnp.reshape(b_out, (1, T)),
    )


# speedup vs baseline: 7.1068x; 1.2511x over previous
import jax
import jax.numpy as jnp
from jax import lax
from jax.experimental import pallas as pl
from jax.experimental.pallas import tpu as pltpu
from jax.experimental.pallas import tpu_sc as plsc

N = 10000
E = 320000
D = 128
DE = 16
K = 8
G = 128
T = 10

NC = 2
NS = 16
CHUNK = 80
EPT = E // (NC * NS)
NCHUNK = EPT // CHUNK
NPAD = 10240
RPT = NPAD // NS
XZ = 64
SZ = 160


def _isnan0(v):
    return jnp.where(jnp.isnan(v), jnp.float32(0.0), v)


def _sqrt_sc(z):
    i = plsc.bitcast(z, jnp.int32)
    i = jnp.int32(0x5F3759DF) - (i >> 1)
    y = plsc.bitcast(i, jnp.float32)
    for _ in range(3):
        y = y * (jnp.float32(1.5) - jnp.float32(0.5) * z * y * y)
    return z * y


def _sc_body(x_hbm, u_hbm, src_hbm, dst_hbm, ea_hbm,
             out_x, out_ea, out_pe,
             acc_x, acc_ea,
             src_idx, dst_idx, x_rows, ea_buf, us, ud, pe_acc,
             zb_x, zb_s, sem, sem_s, sem_d, sem_l1, sem_l2, sem_l3,
             sem_ea):
    cid = lax.axis_index("c")
    sid = lax.axis_index("s")
    wid = cid * NS + sid

    zero16 = jnp.zeros((16,), jnp.float32)

    def zrow_x(r, carry):
        for cc in range(D // 16):
            zb_x[r, pl.ds(cc * 16, 16)] = zero16
        return carry
    lax.fori_loop(0, XZ, zrow_x, None)

    def zrow_s(r, carry):
        zb_s[r, :] = zero16
        return carry
    lax.fori_loop(0, SZ, zrow_s, None)

    def zrow_pe(r, carry):
        pe_acc[pl.ds(r * 16, 16)] = zero16
        return carry
    lax.fori_loop(0, N // 16, zrow_pe, None)

    for j in range(RPT // XZ):
        pltpu.sync_copy(zb_x, acc_x.at[pl.ds(sid * RPT + j * XZ, XZ)])
    for j in range(RPT // SZ):
        pltpu.sync_copy(zb_s, acc_ea.at[pl.ds(sid * RPT + j * SZ, SZ)])
    plsc.subcore_barrier()

    lanes = lax.iota(jnp.int32, 16)

    def chunk_body(c, carry):
        e0 = wid * EPT + c * CHUNK
        l1 = pltpu.async_copy(src_hbm.at[pl.ds(e0, CHUNK)], src_idx, sem_l1)
        l2 = pltpu.async_copy(dst_hbm.at[pl.ds(e0, CHUNK)], dst_idx, sem_l2)
        l3 = pltpu.async_copy(ea_hbm.at[pl.ds(e0, CHUNK)], ea_buf, sem_l3)
        l1.wait()
        cx = pltpu.async_copy(x_hbm.at[src_idx], x_rows, sem)
        cs = pltpu.async_copy(u_hbm.at[src_idx], us, sem_s)
        l2.wait()
        cd = pltpu.async_copy(u_hbm.at[dst_idx], ud, sem_d)
        l3.wait()
        cea = pltpu.async_copy(ea_buf, acc_ea.at[dst_idx], sem_ea, add=True)
        cs.wait()
        cd.wait()

        for g in range(CHUNK // 16):
            rows = lanes + g * 16
            acc = jnp.zeros((16,), jnp.float32)
            for k in range(K):
                kk = jnp.full((16,), k, jnp.int32)
                a = _isnan0(plsc.load_gather(us, [rows, kk]))
                b = _isnan0(plsc.load_gather(ud, [rows, kk]))
                d = a - b
                acc = acc + d * d
            pe = _sqrt_sc(acc + jnp.float32(1e-12))
            dv = dst_idx[pl.ds(g * 16, 16)]
            plsc.addupdate_scatter(pe_acc, [dv], pe)

        cx.wait()
        pltpu.sync_copy(x_rows, acc_x.at[dst_idx], add=True)
        cea.wait()
        return carry

    lax.fori_loop(0, NCHUNK, chunk_body, None)
    plsc.subcore_barrier()

    base = cid * NPAD + sid * RPT
    for j in range(RPT // XZ):
        pltpu.sync_copy(acc_x.at[pl.ds(sid * RPT + j * XZ, XZ)], zb_x)
        pltpu.sync_copy(zb_x, out_x.at[pl.ds(base + j * XZ, XZ)])
    for j in range(RPT // SZ):
        pltpu.sync_copy(acc_ea.at[pl.ds(sid * RPT + j * SZ, SZ)], zb_s)
        pltpu.sync_copy(zb_s, out_ea.at[pl.ds(base + j * SZ, SZ)])
    pltpu.sync_copy(pe_acc, out_pe.at[pl.ds(wid * N, N)])


_sc_edges = pl.kernel(
    _sc_body,
    out_type=(
        jax.ShapeDtypeStruct((NC * NPAD, D), jnp.float32),
        jax.ShapeDtypeStruct((NC * NPAD, DE), jnp.float32),
        jax.ShapeDtypeStruct((NC * NS * N,), jnp.float32),
    ),
    mesh=plsc.VectorSubcoreMesh(core_axis_name="c", subcore_axis_name="s"),
    compiler_params=pltpu.CompilerParams(
        needs_layout_passes=False, use_tc_tiling_on_sc=False),
    scratch_types=(
        pltpu.VMEM_SHARED((NPAD, D), jnp.float32),
        pltpu.VMEM_SHARED((NPAD, DE), jnp.float32),
        pltpu.VMEM((CHUNK,), jnp.int32),
        pltpu.VMEM((CHUNK,), jnp.int32),
        pltpu.VMEM((CHUNK, D), jnp.float32),
        pltpu.VMEM((CHUNK, DE), jnp.float32),
        pltpu.VMEM((CHUNK, 2 * K), jnp.float32),
        pltpu.VMEM((CHUNK, 2 * K), jnp.float32),
        pltpu.VMEM((N,), jnp.float32),
        pltpu.VMEM((XZ, D), jnp.float32),
        pltpu.VMEM((SZ, DE), jnp.float32),
        pltpu.SemaphoreType.DMA,
        pltpu.SemaphoreType.DMA,
        pltpu.SemaphoreType.DMA,
        pltpu.SemaphoreType.DMA,
        pltpu.SemaphoreType.DMA,
        pltpu.SemaphoreType.DMA,
        pltpu.SemaphoreType.DMA,
    ),
)


ROWS = 1024
NB = NPAD // ROWS


def _tc_body(x_ref, px0, px1, pea0, pea1, ppe_ref, s_ref, batch_ref,
             we_ref, bpe_ref, ws_ref, wn_ref, wo_ref, bo_ref,
             out_ref, gsum, gcnt):
    i = pl.program_id(0)

    @pl.when(i == 0)
    def _init():
        gsum[...] = jnp.zeros((G, D), jnp.float32)
        gcnt[...] = jnp.zeros((G, 1), jnp.float32)

    agg = (px0[...] + px1[...]
           + jnp.dot(pea0[...] + pea1[...], we_ref[...],
                     preferred_element_type=jnp.float32,
                     precision=lax.Precision.HIGHEST)
           + jnp.sum(ppe_ref[...], axis=1, keepdims=True) * bpe_ref[...])
    pre = (jnp.dot(x_ref[...], ws_ref[...], preferred_element_type=jnp.float32,
                     precision=lax.Precision.HIGHEST)
           + jnp.dot(agg, wn_ref[...], preferred_element_type=jnp.float32,
                     precision=lax.Precision.HIGHEST))
    enc = jnp.maximum(pre * s_ref[...], 0.0)

    b2 = jnp.reshape(batch_ref[...], (1, ROWS))
    oht = (lax.broadcasted_iota(jnp.int32, (G, ROWS), 0) == b2
           ).astype(jnp.float32)
    gsum[...] += jnp.dot(oht, enc, preferred_element_type=jnp.float32,
                     precision=lax.Precision.HIGHEST)
    gcnt[...] += jnp.sum(oht, axis=1, keepdims=True)

    @pl.when(i == NB - 1)
    def _fin():
        rep = gsum[...] / jnp.maximum(gcnt[...], 1.0)
        out_ref[...] = (jnp.dot(rep, wo_ref[...],
                                preferred_element_type=jnp.float32,
                     precision=lax.Precision.HIGHEST)
                        + bo_ref[...])


def _row_spec(cols):
    return pl.BlockSpec((ROWS, cols), lambda i: (i, 0))


def _row_spec2(cols):
    return pl.BlockSpec((ROWS, cols), lambda i: (i + NB, 0))


def _full_spec(r, c):
    return pl.BlockSpec((r, c), lambda i: (0, 0))


_tc_dense = pl.pallas_call(
    _tc_body,
    grid=(NB,),
    in_specs=[
        _row_spec(D), _row_spec(D), _row_spec2(D),
        _row_spec(DE), _row_spec2(DE), _row_spec(NC * NS),
        _row_spec(1),
        pl.BlockSpec((1, 1, ROWS), lambda i: (i, 0, 0)),
        _full_spec(DE, D), _full_spec(1, D), _full_spec(D, D),
        _full_spec(D, D), _full_spec(D, T), _full_spec(1, T),
    ],
    out_specs=pl.BlockSpec((G, T), lambda i: (0, 0)),
    out_shape=jax.ShapeDtypeStruct((G, T), jnp.float32),
    scratch_shapes=[
        pltpu.VMEM((G, D), jnp.float32),
        pltpu.VMEM((G, 1), jnp.float32),
    ],
)


@jax.jit
def kernel(x, edge_index, edge_attr, snorm_n, EigVals, EigVecs, batch,
           W_edge, b_pe, W_self, W_nbr, W_out, b_out):
    src = edge_index[0]
    dst = edge_index[1]
    u_pad = jnp.pad(EigVecs, ((0, 0), (0, K)))
    part_x, part_ea, part_pe = _sc_edges(x, u_pad, src, dst, edge_attr)
    pad_n = NPAD - N
    x_p = jnp.pad(x, ((0, pad_n), (0, 0)))
    s_p = jnp.pad(snorm_n, ((0, pad_n), (0, 0)))
    pe_t = jnp.pad(jnp.transpose(jnp.reshape(part_pe, (NC * NS, N))),
                   ((0, pad_n), (0, 0)))
    batch3 = jnp.reshape(
        jnp.pad(batch, (0, pad_n), constant_values=G), (NB, 1, ROWS))
    return _tc_dense(
        x_p, part_x, part_x,
        part_ea, part_ea, pe_t,
        s_p, batch3,
        W_edge, jnp.reshape(b_pe, (1, D)), W_self, W_nbr,
        W_out, jnp.reshape(b_out, (1, T)),
    )


# speedup vs baseline: 7.3290x; 1.0313x over previous
import jax
import jax.numpy as jnp
from jax import lax
from jax.experimental import pallas as pl
from jax.experimental.pallas import tpu as pltpu
from jax.experimental.pallas import tpu_sc as plsc

N = 10000
E = 320000
D = 128
DE = 16
K = 8
G = 128
T = 10

NC = 2
NS = 16
CHUNK = 80
EPT = E // (NC * NS)
NCHUNK = EPT // CHUNK
NPAD = 10240
RPT = NPAD // NS
XZ = 64
SZ = 160


def _isnan0(v):
    return jnp.where(jnp.isnan(v), jnp.float32(0.0), v)


def _sqrt_sc(z):
    i = plsc.bitcast(z, jnp.int32)
    i = jnp.int32(0x5F3759DF) - (i >> 1)
    y = plsc.bitcast(i, jnp.float32)
    for _ in range(3):
        y = y * (jnp.float32(1.5) - jnp.float32(0.5) * z * y * y)
    return z * y


def _sc_body(x_hbm, u_hbm, src_hbm, dst_hbm, ea_hbm,
             out_x, out_ea, out_pe,
             acc_x, acc_ea,
             src_idx, dst_idx, x_rows, ea_buf, us, ud, pe_acc,
             zb_x, zb_s, sem, sem_s, sem_d, sem_l1, sem_l2, sem_l3,
             sem_ea):
    cid = lax.axis_index("c")
    sid = lax.axis_index("s")
    wid = cid * NS + sid

    zero16 = jnp.zeros((16,), jnp.float32)

    def zrow_x(r, carry):
        for cc in range(D // 16):
            zb_x[r, pl.ds(cc * 16, 16)] = zero16
        return carry
    lax.fori_loop(0, XZ, zrow_x, None)

    def zrow_s(r, carry):
        zb_s[r, :] = zero16
        return carry
    lax.fori_loop(0, SZ, zrow_s, None)

    def zrow_pe(r, carry):
        pe_acc[pl.ds(r * 16, 16)] = zero16
        return carry
    lax.fori_loop(0, N // 16, zrow_pe, None)

    for j in range(RPT // XZ):
        pltpu.sync_copy(zb_x, acc_x.at[pl.ds(sid * RPT + j * XZ, XZ)])
    for j in range(RPT // SZ):
        pltpu.sync_copy(zb_s, acc_ea.at[pl.ds(sid * RPT + j * SZ, SZ)])
    plsc.subcore_barrier()

    lanes = lax.iota(jnp.int32, 16)

    def chunk_body(c, carry):
        e0 = wid * EPT + c * CHUNK
        l1 = pltpu.async_copy(src_hbm.at[pl.ds(e0, CHUNK)], src_idx, sem_l1)
        l2 = pltpu.async_copy(dst_hbm.at[pl.ds(e0, CHUNK)], dst_idx, sem_l2)
        l3 = pltpu.async_copy(ea_hbm.at[pl.ds(e0, CHUNK)], ea_buf, sem_l3)
        l1.wait()
        cx = pltpu.async_copy(x_hbm.at[src_idx], x_rows, sem)
        cs = pltpu.async_copy(u_hbm.at[src_idx], us, sem_s)
        l2.wait()
        cd = pltpu.async_copy(u_hbm.at[dst_idx], ud, sem_d)
        l3.wait()
        cea = pltpu.async_copy(ea_buf, acc_ea.at[dst_idx], sem_ea, add=True)
        cx.wait()
        csx = pltpu.async_copy(x_rows, acc_x.at[dst_idx], sem, add=True)
        cs.wait()
        cd.wait()

        for g in range(CHUNK // 16):
            rows = lanes + g * 16
            acc = jnp.zeros((16,), jnp.float32)
            for k in range(K):
                kk = jnp.full((16,), k, jnp.int32)
                a = _isnan0(plsc.load_gather(us, [rows, kk]))
                b = _isnan0(plsc.load_gather(ud, [rows, kk]))
                d = a - b
                acc = acc + d * d
            pe = _sqrt_sc(acc + jnp.float32(1e-12))
            dv = dst_idx[pl.ds(g * 16, 16)]
            plsc.addupdate_scatter(pe_acc, [dv], pe)

        csx.wait()
        cea.wait()
        return carry

    lax.fori_loop(0, NCHUNK, chunk_body, None)
    plsc.subcore_barrier()

    base = cid * NPAD + sid * RPT
    for j in range(RPT // XZ):
        pltpu.sync_copy(acc_x.at[pl.ds(sid * RPT + j * XZ, XZ)], zb_x)
        pltpu.sync_copy(zb_x, out_x.at[pl.ds(base + j * XZ, XZ)])
    for j in range(RPT // SZ):
        pltpu.sync_copy(acc_ea.at[pl.ds(sid * RPT + j * SZ, SZ)], zb_s)
        pltpu.sync_copy(zb_s, out_ea.at[pl.ds(base + j * SZ, SZ)])
    pltpu.sync_copy(pe_acc, out_pe.at[pl.ds(wid * N, N)])


_sc_edges = pl.kernel(
    _sc_body,
    out_type=(
        jax.ShapeDtypeStruct((NC * NPAD, D), jnp.float32),
        jax.ShapeDtypeStruct((NC * NPAD, DE), jnp.float32),
        jax.ShapeDtypeStruct((NC * NS * N,), jnp.float32),
    ),
    mesh=plsc.VectorSubcoreMesh(core_axis_name="c", subcore_axis_name="s"),
    compiler_params=pltpu.CompilerParams(
        needs_layout_passes=False, use_tc_tiling_on_sc=False),
    scratch_types=(
        pltpu.VMEM_SHARED((NPAD, D), jnp.float32),
        pltpu.VMEM_SHARED((NPAD, DE), jnp.float32),
        pltpu.VMEM((CHUNK,), jnp.int32),
        pltpu.VMEM((CHUNK,), jnp.int32),
        pltpu.VMEM((CHUNK, D), jnp.float32),
        pltpu.VMEM((CHUNK, DE), jnp.float32),
        pltpu.VMEM((CHUNK, K), jnp.float32),
        pltpu.VMEM((CHUNK, K), jnp.float32),
        pltpu.VMEM((N,), jnp.float32),
        pltpu.VMEM((XZ, D), jnp.float32),
        pltpu.VMEM((SZ, DE), jnp.float32),
        pltpu.SemaphoreType.DMA,
        pltpu.SemaphoreType.DMA,
        pltpu.SemaphoreType.DMA,
        pltpu.SemaphoreType.DMA,
        pltpu.SemaphoreType.DMA,
        pltpu.SemaphoreType.DMA,
        pltpu.SemaphoreType.DMA,
    ),
)


ROWS = 1024
NB = NPAD // ROWS


def _tc_body(x_ref, px0, px1, pea0, pea1, ppe_ref, s_ref, batch_ref,
             we_ref, bpe_ref, ws_ref, wn_ref, wo_ref, bo_ref,
             out_ref, gsum, gcnt):
    i = pl.program_id(0)

    @pl.when(i == 0)
    def _init():
        gsum[...] = jnp.zeros((G, D), jnp.float32)
        gcnt[...] = jnp.zeros((G, 1), jnp.float32)

    agg = (px0[...] + px1[...]
           + jnp.dot(pea0[...] + pea1[...], we_ref[...],
                     preferred_element_type=jnp.float32,
                     precision=lax.Precision.HIGHEST)
           + jnp.sum(ppe_ref[...], axis=1, keepdims=True) * bpe_ref[...])
    pre = (jnp.dot(x_ref[...], ws_ref[...], preferred_element_type=jnp.float32,
                     precision=lax.Precision.HIGHEST)
           + jnp.dot(agg, wn_ref[...], preferred_element_type=jnp.float32,
                     precision=lax.Precision.HIGHEST))
    enc = jnp.maximum(pre * s_ref[...], 0.0)

    b2 = jnp.reshape(batch_ref[...], (1, ROWS))
    oht = (lax.broadcasted_iota(jnp.int32, (G, ROWS), 0) == b2
           ).astype(jnp.float32)
    gsum[...] += jnp.dot(oht, enc, preferred_element_type=jnp.float32,
                     precision=lax.Precision.HIGHEST)
    gcnt[...] += jnp.sum(oht, axis=1, keepdims=True)

    @pl.when(i == NB - 1)
    def _fin():
        rep = gsum[...] / jnp.maximum(gcnt[...], 1.0)
        out_ref[...] = (jnp.dot(rep, wo_ref[...],
                                preferred_element_type=jnp.float32,
                     precision=lax.Precision.HIGHEST)
                        + bo_ref[...])


def _row_spec(cols):
    return pl.BlockSpec((ROWS, cols), lambda i: (i, 0))


def _row_spec2(cols):
    return pl.BlockSpec((ROWS, cols), lambda i: (i + NB, 0))


def _full_spec(r, c):
    return pl.BlockSpec((r, c), lambda i: (0, 0))


_tc_dense = pl.pallas_call(
    _tc_body,
    grid=(NB,),
    in_specs=[
        _row_spec(D), _row_spec(D), _row_spec2(D),
        _row_spec(DE), _row_spec2(DE), _row_spec(NC * NS),
        _row_spec(1),
        pl.BlockSpec((1, 1, ROWS), lambda i: (i, 0, 0)),
        _full_spec(DE, D), _full_spec(1, D), _full_spec(D, D),
        _full_spec(D, D), _full_spec(D, T), _full_spec(1, T),
    ],
    out_specs=pl.BlockSpec((G, T), lambda i: (0, 0)),
    out_shape=jax.ShapeDtypeStruct((G, T), jnp.float32),
    scratch_shapes=[
        pltpu.VMEM((G, D), jnp.float32),
        pltpu.VMEM((G, 1), jnp.float32),
    ],
)


@jax.jit
def kernel(x, edge_index, edge_attr, snorm_n, EigVals, EigVecs, batch,
           W_edge, b_pe, W_self, W_nbr, W_out, b_out):
    src = edge_index[0]
    dst = edge_index[1]
    part_x, part_ea, part_pe = _sc_edges(x, EigVecs, src, dst, edge_attr)
    pad_n = NPAD - N
    x_p = jnp.pad(x, ((0, pad_n), (0, 0)))
    s_p = jnp.pad(snorm_n, ((0, pad_n), (0, 0)))
    pe_t = jnp.pad(jnp.transpose(jnp.reshape(part_pe, (NC * NS, N))),
                   ((0, pad_n), (0, 0)))
    batch3 = jnp.reshape(
        jnp.pad(batch, (0, pad_n), constant_values=G), (NB, 1, ROWS))
    return _tc_dense(
        x_p, part_x, part_x,
        part_ea, part_ea, pe_t,
        s_p, batch3,
        W_edge, jnp.reshape(b_pe, (1, D)), W_self, W_nbr,
        W_out, jnp.reshape(b_out, (1, T)),
    )


# speedup vs baseline: 7.4300x; 1.0138x over previous
import jax
import jax.numpy as jnp
from jax import lax
from jax.experimental import pallas as pl
from jax.experimental.pallas import tpu as pltpu
from jax.experimental.pallas import tpu_sc as plsc

N = 10000
E = 320000
D = 128
DE = 16
K = 8
G = 128
T = 10

NC = 2
NS = 16
CHUNK = 80
EPT = E // (NC * NS)
NCHUNK = EPT // CHUNK
NPAD = 10240
RPT = NPAD // NS
XZ = 64
SZ = 160


def _isnan0(v):
    return jnp.where(jnp.isnan(v), jnp.float32(0.0), v)


def _sqrt_sc(z):
    i = plsc.bitcast(z, jnp.int32)
    i = jnp.int32(0x5F3759DF) - (i >> 1)
    y = plsc.bitcast(i, jnp.float32)
    for _ in range(3):
        y = y * (jnp.float32(1.5) - jnp.float32(0.5) * z * y * y)
    return z * y


def _sc_body(x_hbm, u_hbm, src_hbm, dst_hbm, ea_hbm,
             out_x, out_ea, out_pe,
             acc_x, acc_ea,
             src_idx, dst_idx, x_rows, ea_buf, us, ud, pe_acc, zbt,
             zb_x, zb_s, sem, sem_s, sem_d, sem_l1, sem_l2, sem_l3,
             sem_ea):
    cid = lax.axis_index("c")
    sid = lax.axis_index("s")
    wid = cid * NS + sid

    zero16 = jnp.zeros((16,), jnp.float32)

    def zrow_x(r, carry):
        for cc in range(D // 16):
            zb_x[r, pl.ds(cc * 16, 16)] = zero16
        return carry
    lax.fori_loop(0, XZ, zrow_x, None)

    def zrow_s(r, carry):
        zb_s[r, :] = zero16
        return carry
    lax.fori_loop(0, SZ, zrow_s, None)

    def zrow_pe(r, carry):
        pe_acc[pl.ds(r * 16, 16)] = zero16
        return carry
    lax.fori_loop(0, N // 16, zrow_pe, None)

    for r in range((NPAD - N) // 16):
        zbt[pl.ds(r * 16, 16)] = zero16

    for j in range(RPT // XZ):
        pltpu.sync_copy(zb_x, acc_x.at[pl.ds(sid * RPT + j * XZ, XZ)])
    for j in range(RPT // SZ):
        pltpu.sync_copy(zb_s, acc_ea.at[pl.ds(sid * RPT + j * SZ, SZ)])
    plsc.subcore_barrier()

    lanes = lax.iota(jnp.int32, 16)

    def chunk_body(c, carry):
        e0 = wid * EPT + c * CHUNK
        l1 = pltpu.async_copy(src_hbm.at[pl.ds(e0, CHUNK)], src_idx, sem_l1)
        l2 = pltpu.async_copy(dst_hbm.at[pl.ds(e0, CHUNK)], dst_idx, sem_l2)
        l3 = pltpu.async_copy(ea_hbm.at[pl.ds(e0, CHUNK)], ea_buf, sem_l3)
        l1.wait()
        cx = pltpu.async_copy(x_hbm.at[src_idx], x_rows, sem)
        cs = pltpu.async_copy(u_hbm.at[src_idx], us, sem_s)
        l2.wait()
        cd = pltpu.async_copy(u_hbm.at[dst_idx], ud, sem_d)
        l3.wait()
        cea = pltpu.async_copy(ea_buf, acc_ea.at[dst_idx], sem_ea, add=True)
        cx.wait()
        csx = pltpu.async_copy(x_rows, acc_x.at[dst_idx], sem, add=True)
        cs.wait()
        cd.wait()

        for g in range(CHUNK // 16):
            rows = lanes + g * 16
            acc = jnp.zeros((16,), jnp.float32)
            for k in range(K):
                kk = jnp.full((16,), k, jnp.int32)
                a = _isnan0(plsc.load_gather(us, [rows, kk]))
                b = _isnan0(plsc.load_gather(ud, [rows, kk]))
                d = a - b
                acc = acc + d * d
            pe = _sqrt_sc(acc + jnp.float32(1e-12))
            dv = dst_idx[pl.ds(g * 16, 16)]
            plsc.addupdate_scatter(pe_acc, [dv], pe)

        csx.wait()
        cea.wait()
        return carry

    lax.fori_loop(0, NCHUNK, chunk_body, None)
    plsc.subcore_barrier()

    base = cid * NPAD + sid * RPT
    for j in range(RPT // XZ):
        pltpu.sync_copy(acc_x.at[pl.ds(sid * RPT + j * XZ, XZ)], zb_x)
        pltpu.sync_copy(zb_x, out_x.at[pl.ds(base + j * XZ, XZ)])
    for j in range(RPT // SZ):
        pltpu.sync_copy(acc_ea.at[pl.ds(sid * RPT + j * SZ, SZ)], zb_s)
        pltpu.sync_copy(zb_s, out_ea.at[pl.ds(base + j * SZ, SZ)])
    pltpu.sync_copy(pe_acc, out_pe.at[wid, pl.ds(0, N)])
    pltpu.sync_copy(zbt, out_pe.at[wid, pl.ds(N, NPAD - N)])


_sc_edges = pl.kernel(
    _sc_body,
    out_type=(
        jax.ShapeDtypeStruct((NC * NPAD, D), jnp.float32),
        jax.ShapeDtypeStruct((NC * NPAD, DE), jnp.float32),
        jax.ShapeDtypeStruct((NC * NS, NPAD), jnp.float32),
    ),
    mesh=plsc.VectorSubcoreMesh(core_axis_name="c", subcore_axis_name="s"),
    compiler_params=pltpu.CompilerParams(
        needs_layout_passes=False, use_tc_tiling_on_sc=False),
    scratch_types=(
        pltpu.VMEM_SHARED((NPAD, D), jnp.float32),
        pltpu.VMEM_SHARED((NPAD, DE), jnp.float32),
        pltpu.VMEM((CHUNK,), jnp.int32),
        pltpu.VMEM((CHUNK,), jnp.int32),
        pltpu.VMEM((CHUNK, D), jnp.float32),
        pltpu.VMEM((CHUNK, DE), jnp.float32),
        pltpu.VMEM((CHUNK, K), jnp.float32),
        pltpu.VMEM((CHUNK, K), jnp.float32),
        pltpu.VMEM((N,), jnp.float32),
        pltpu.VMEM((NPAD - N,), jnp.float32),
        pltpu.VMEM((XZ, D), jnp.float32),
        pltpu.VMEM((SZ, DE), jnp.float32),
        pltpu.SemaphoreType.DMA,
        pltpu.SemaphoreType.DMA,
        pltpu.SemaphoreType.DMA,
        pltpu.SemaphoreType.DMA,
        pltpu.SemaphoreType.DMA,
        pltpu.SemaphoreType.DMA,
        pltpu.SemaphoreType.DMA,
    ),
)


ROWS = 1024
NB = NPAD // ROWS


def _tc_body(x_ref, px0, px1, pea0, pea1, ppe_ref, s_ref, batch_ref,
             we_ref, bpe_ref, ws_ref, wn_ref, wo_ref, bo_ref,
             out_ref, gsum, gcnt):
    i = pl.program_id(0)

    @pl.when(i == 0)
    def _init():
        gsum[...] = jnp.zeros((G, D), jnp.float32)
        gcnt[...] = jnp.zeros((G, 1), jnp.float32)

    agg = (px0[...] + px1[...]
           + jnp.dot(pea0[...] + pea1[...], we_ref[...],
                     preferred_element_type=jnp.float32,
                     precision=lax.Precision.HIGHEST)
           + lax.dot_general(ppe_ref[...], jnp.ones((NC * NS, 1), jnp.float32),
                             (((0,), (0,)), ((), ())),
                             preferred_element_type=jnp.float32) * bpe_ref[...])
    pre = (jnp.dot(x_ref[...], ws_ref[...], preferred_element_type=jnp.float32,
                     precision=lax.Precision.HIGHEST)
           + jnp.dot(agg, wn_ref[...], preferred_element_type=jnp.float32,
                     precision=lax.Precision.HIGHEST))
    enc = jnp.maximum(pre * s_ref[...], 0.0)

    b2 = jnp.reshape(batch_ref[...], (1, ROWS))
    oht = (lax.broadcasted_iota(jnp.int32, (G, ROWS), 0) == b2
           ).astype(jnp.float32)
    gsum[...] += jnp.dot(oht, enc, preferred_element_type=jnp.float32,
                     precision=lax.Precision.HIGHEST)
    gcnt[...] += jnp.sum(oht, axis=1, keepdims=True)

    @pl.when(i == NB - 1)
    def _fin():
        rep = gsum[...] / jnp.maximum(gcnt[...], 1.0)
        out_ref[...] = (jnp.dot(rep, wo_ref[...],
                                preferred_element_type=jnp.float32,
                     precision=lax.Precision.HIGHEST)
                        + bo_ref[...])


def _row_spec(cols):
    return pl.BlockSpec((ROWS, cols), lambda i: (i, 0))


def _row_spec2(cols):
    return pl.BlockSpec((ROWS, cols), lambda i: (i + NB, 0))


def _full_spec(r, c):
    return pl.BlockSpec((r, c), lambda i: (0, 0))


_tc_dense = pl.pallas_call(
    _tc_body,
    grid=(NB,),
    in_specs=[
        _row_spec(D), _row_spec(D), _row_spec2(D),
        _row_spec(DE), _row_spec2(DE),
        pl.BlockSpec((NC * NS, ROWS), lambda i: (0, i)),
        _row_spec(1),
        pl.BlockSpec((1, 1, ROWS), lambda i: (i, 0, 0)),
        _full_spec(DE, D), _full_spec(1, D), _full_spec(D, D),
        _full_spec(D, D), _full_spec(D, T), _full_spec(1, T),
    ],
    out_specs=pl.BlockSpec((G, T), lambda i: (0, 0)),
    out_shape=jax.ShapeDtypeStruct((G, T), jnp.float32),
    scratch_shapes=[
        pltpu.VMEM((G, D), jnp.float32),
        pltpu.VMEM((G, 1), jnp.float32),
    ],
)


@jax.jit
def kernel(x, edge_index, edge_attr, snorm_n, EigVals, EigVecs, batch,
           W_edge, b_pe, W_self, W_nbr, W_out, b_out):
    src = edge_index[0]
    dst = edge_index[1]
    part_x, part_ea, part_pe = _sc_edges(x, EigVecs, src, dst, edge_attr)
    pad_n = NPAD - N
    x_p = jnp.pad(x, ((0, pad_n), (0, 0)))
    s_p = jnp.pad(snorm_n, ((0, pad_n), (0, 0)))
    batch3 = jnp.reshape(
        jnp.pad(batch, (0, pad_n), constant_values=G), (NB, 1, ROWS))
    return _tc_dense(
        x_p, part_x, part_x,
        part_ea, part_ea, part_pe,
        s_p, batch3,
        W_edge, jnp.reshape(b_pe, (1, D)), W_self, W_nbr,
        W_out, jnp.reshape(b_out, (1, T)),
    )
